# Initial kernel scaffold; baseline (speedup 1.0000x reference)
import jax
import jax.numpy as jnp
from jax.experimental import pallas as pl


def kernel(seq, adj, W, bias, alpha):
    raise NotImplementedError("write your pallas kernel here")


# speedup vs baseline: 1.0420x; 1.0420x over previous
import jax
import jax.numpy as jnp
from jax.experimental import pallas as pl
from jax.experimental.pallas import tpu as pltpu


def _gcn_kernel(seq_ref, w_ref, adj_ref, bias_ref, alpha_ref, out_ref, fts_ref):
    @pl.when(pl.program_id(0) == 0)
    def _():
        fts_ref[...] = jax.lax.dot_general(
            seq_ref[...], w_ref[...],
            dimension_numbers=(((1,), (1,)), ((), ())),
            preferred_element_type=jnp.float32,
        )

    acc = jax.lax.dot_general(
        adj_ref[...], fts_ref[...],
        dimension_numbers=(((1,), (0,)), ((), ())),
        preferred_element_type=jnp.float32,
    )
    acc = acc + bias_ref[...]
    alpha = alpha_ref[0]
    out_ref[...] = jnp.where(acc > 0, acc, alpha * acc)


def kernel(seq, adj, W, bias, alpha):
    _, n, d_in = seq.shape
    d_out = W.shape[0]
    seq2 = seq.reshape(n, d_in)
    adj2 = adj.reshape(n, n)
    bias2 = bias.reshape(1, d_out)
    alpha2 = alpha.reshape(1)

    blk = 400
    grid = (n // blk,)
    out = pl.pallas_call(
        _gcn_kernel,
        grid=grid,
        in_specs=[
            pl.BlockSpec((n, d_in), lambda i: (0, 0)),
            pl.BlockSpec((d_out, d_in), lambda i: (0, 0)),
            pl.BlockSpec((blk, n), lambda i: (i, 0)),
            pl.BlockSpec((1, d_out), lambda i: (0, 0)),
            pl.BlockSpec(memory_space=pltpu.SMEM),
        ],
        out_specs=pl.BlockSpec((blk, d_out), lambda i: (i, 0)),
        out_shape=jax.ShapeDtypeStruct((n, d_out), jnp.float32),
        scratch_shapes=[pltpu.VMEM((n, d_out), jnp.float32)],
    )(seq2, W, adj2, bias2, alpha2)
    return out.reshape(1, n, d_out)
